# baseline (speedup 1.0000x reference)
import functools

import jax
import jax.numpy as jnp
from jax import lax
from jax.experimental import pallas as pl
from jax.experimental.pallas import tpu as pltpu
from jax.experimental.pallas import tpu_sc as plsc

TOP_K = 8
N_EXP = 64
SCALING = 2.5
N_TOK = 32768
NC, NS, L = 2, 16, 16
NW = NC * NS
TPW = N_TOK // NW
UNROLL_T = 2
EXPERT_UNROLL = 4
N_STEP = TPW // (L * UNROLL_T)

_mesh = plsc.VectorSubcoreMesh(core_axis_name="c", subcore_axis_name="s")


@functools.partial(
    pl.kernel,
    out_type=(
        jax.ShapeDtypeStruct((N_TOK * TOP_K,), jnp.float32),
        jax.ShapeDtypeStruct((N_TOK * TOP_K,), jnp.int32),
        jax.ShapeDtypeStruct((NW, N_EXP), jnp.int32),
    ),
    mesh=_mesh,
    compiler_params=pltpu.CompilerParams(needs_layout_passes=False),
    scratch_types=(
        pltpu.VMEM((TPW * N_EXP,), jnp.float32),
        pltpu.VMEM((TPW * TOP_K,), jnp.float32),
        pltpu.VMEM((TPW * TOP_K,), jnp.int32),
        pltpu.VMEM((N_EXP,), jnp.float32),
        pltpu.VMEM((N_EXP, UNROLL_T * L), jnp.int32),
        pltpu.VMEM((N_EXP,), jnp.int32),
    ),
)
def _router(logits_hbm, bias_hbm, w_hbm, i_hbm, h_hbm,
            x_v, w_v, i_v, b_v, hist_v, hred_v):
    c = lax.axis_index("c")
    s = lax.axis_index("s")
    wid = s * NC + c
    base = wid * TPW

    iota = jnp.arange(L, dtype=jnp.int32)
    zeros_i = jnp.zeros((L,), jnp.int32)
    ones_i = jnp.ones((L,), jnp.int32)

    pltpu.sync_copy(bias_hbm, b_v)
    pltpu.sync_copy(logits_hbm.at[pl.ds(base * N_EXP, TPW * N_EXP)], x_v)

    def _zero_body(e, _):
        for u in range(UNROLL_T):
            hist_v[e, pl.ds(u * L, L)] = zeros_i
        return 0
    lax.fori_loop(0, N_EXP, _zero_body, 0)

    neg_inf = jnp.full((L,), -jnp.inf, jnp.float32)

    def step(g0, _):
        xbase = []
        for u in range(UNROLL_T):
            tok = (g0 * UNROLL_T + u) * L + iota
            xbase.append(tok * N_EXP)

        def expert_body(eb, carry):
            ms, mis = carry
            new_ms = [list(ms[u]) for u in range(UNROLL_T)]
            new_mis = [list(mis[u]) for u in range(UNROLL_T)]
            for eu in range(EXPERT_UNROLL):
                e = eb * EXPERT_UNROLL + eu
                e_splat = zeros_i + e
                bias_e = plsc.load_gather(b_v, [e_splat])
                for u in range(UNROLL_T):
                    x = plsc.load_gather(x_v, [xbase[u] + e])
                    v = 1.0 / (1.0 + jnp.exp(-x)) + bias_e
                    vi = e_splat
                    m = new_ms[u]
                    mi = new_mis[u]
                    for j in range(TOP_K):
                        b = v > m[j]
                        m[j], v = jnp.where(b, v, m[j]), jnp.where(b, m[j], v)
                        mi[j], vi = jnp.where(b, vi, mi[j]), jnp.where(b, mi[j], vi)
            return (tuple(tuple(r) for r in new_ms),
                    tuple(tuple(r) for r in new_mis))

        init = (
            tuple(tuple(neg_inf for _ in range(TOP_K)) for _ in range(UNROLL_T)),
            tuple(tuple(zeros_i for _ in range(TOP_K)) for _ in range(UNROLL_T)),
        )
        ms, mis = lax.fori_loop(0, N_EXP // EXPERT_UNROLL, expert_body, init)

        for u in range(UNROLL_T):
            m, mi = ms[u], mis[u]
            sv = [m[j] - plsc.load_gather(b_v, [mi[j]]) for j in range(TOP_K)]
            den = sv[0]
            for j in range(1, TOP_K):
                den = den + sv[j]
            fac = SCALING / (den + 1e-20)
            obase = (g0 * UNROLL_T + u) * L * TOP_K + iota * TOP_K
            for j in range(TOP_K):
                plsc.store_scatter(w_v, [obase + j], sv[j] * fac)
                plsc.store_scatter(i_v, [obase + j], mi[j])
                plsc.addupdate_scatter(hist_v, [mi[j], iota + u * L], ones_i)
        return 0

    lax.fori_loop(0, N_STEP, step, 0)

    pltpu.sync_copy(w_v, w_hbm.at[pl.ds(base * TOP_K, TPW * TOP_K)])
    pltpu.sync_copy(i_v, i_hbm.at[pl.ds(base * TOP_K, TPW * TOP_K)])

    for f in range(N_EXP // L):
        rows = iota + L * f
        acc = plsc.load_gather(hist_v, [rows, zeros_i])
        for u in range(1, UNROLL_T * L):
            acc = acc + plsc.load_gather(hist_v, [rows, zeros_i + u])
        hred_v[pl.ds(L * f, L)] = acc
    pltpu.sync_copy(hred_v, h_hbm.at[wid])


def _hist_sum_body(h_ref, o_ref):
    o_ref[...] = jnp.sum(h_ref[...], axis=0)


_hist_sum = pl.pallas_call(
    _hist_sum_body,
    out_shape=jax.ShapeDtypeStruct((N_EXP,), jnp.int32),
)


def kernel(logits, e_score_correction_bias):
    w_flat, i_flat, h_part = _router(logits.reshape(-1), e_score_correction_bias)
    topk_weight = w_flat.reshape(N_TOK, TOP_K)
    topk_idx = i_flat.reshape(N_TOK, TOP_K)
    tokens_per_expert = _hist_sum(h_part)
    return (logits, topk_weight, topk_idx, tokens_per_expert)


# speedup vs baseline: 1.0894x; 1.0894x over previous
import functools

import jax
import jax.numpy as jnp
from jax import lax
from jax.experimental import pallas as pl
from jax.experimental.pallas import tpu as pltpu
from jax.experimental.pallas import tpu_sc as plsc

TOP_K = 8
N_EXP = 64
SCALING = 2.5
N_TOK = 32768
NC, NS, L = 2, 16, 16
NW = NC * NS
TPW = N_TOK // NW
UNROLL_T = 2
EXPERT_UNROLL = 1
N_STEP = TPW // (L * UNROLL_T)

_mesh = plsc.VectorSubcoreMesh(core_axis_name="c", subcore_axis_name="s")


@functools.partial(
    pl.kernel,
    out_type=(
        jax.ShapeDtypeStruct((N_TOK * TOP_K,), jnp.float32),
        jax.ShapeDtypeStruct((N_TOK * TOP_K,), jnp.int32),
        jax.ShapeDtypeStruct((NW, N_EXP), jnp.int32),
    ),
    mesh=_mesh,
    compiler_params=pltpu.CompilerParams(needs_layout_passes=False),
    scratch_types=(
        pltpu.VMEM((TPW * N_EXP,), jnp.float32),
        pltpu.VMEM((TPW * TOP_K,), jnp.float32),
        pltpu.VMEM((TPW * TOP_K,), jnp.int32),
        pltpu.VMEM((N_EXP,), jnp.float32),
        pltpu.VMEM((N_EXP, UNROLL_T * L), jnp.int32),
        pltpu.VMEM((N_EXP,), jnp.int32),
    ),
)
def _router(logits_hbm, bias_hbm, w_hbm, i_hbm, h_hbm,
            x_v, w_v, i_v, b_v, hist_v, hred_v):
    c = lax.axis_index("c")
    s = lax.axis_index("s")
    wid = s * NC + c
    base = wid * TPW

    iota = jnp.arange(L, dtype=jnp.int32)
    zeros_i = jnp.zeros((L,), jnp.int32)
    ones_i = jnp.ones((L,), jnp.int32)

    pltpu.sync_copy(bias_hbm, b_v)
    pltpu.sync_copy(logits_hbm.at[pl.ds(base * N_EXP, TPW * N_EXP)], x_v)

    def _zero_body(e, _):
        for u in range(UNROLL_T):
            hist_v[e, pl.ds(u * L, L)] = zeros_i
        return 0
    lax.fori_loop(0, N_EXP, _zero_body, 0)

    neg_inf = jnp.full((L,), -jnp.inf, jnp.float32)

    def step(g0, _):
        xbase = []
        for u in range(UNROLL_T):
            tok = (g0 * UNROLL_T + u) * L + iota
            xbase.append(tok * N_EXP)

        def expert_body(eb, carry):
            ms, mis = carry
            new_ms = [list(ms[u]) for u in range(UNROLL_T)]
            new_mis = [list(mis[u]) for u in range(UNROLL_T)]
            for eu in range(EXPERT_UNROLL):
                e = eb * EXPERT_UNROLL + eu
                e_splat = zeros_i + e
                bias_e = plsc.load_gather(b_v, [e_splat])
                for u in range(UNROLL_T):
                    x = plsc.load_gather(x_v, [xbase[u] + e])
                    v = 1.0 / (1.0 + jnp.exp(-x)) + bias_e
                    vi = e_splat
                    m = new_ms[u]
                    mi = new_mis[u]
                    for j in range(TOP_K):
                        b = v > m[j]
                        m[j], v = jnp.where(b, v, m[j]), jnp.where(b, m[j], v)
                        mi[j], vi = jnp.where(b, vi, mi[j]), jnp.where(b, mi[j], vi)
            return (tuple(tuple(r) for r in new_ms),
                    tuple(tuple(r) for r in new_mis))

        init = (
            tuple(tuple(neg_inf for _ in range(TOP_K)) for _ in range(UNROLL_T)),
            tuple(tuple(zeros_i for _ in range(TOP_K)) for _ in range(UNROLL_T)),
        )
        ms, mis = lax.fori_loop(0, N_EXP // EXPERT_UNROLL, expert_body, init)

        for u in range(UNROLL_T):
            m, mi = ms[u], mis[u]
            sv = [m[j] - plsc.load_gather(b_v, [mi[j]]) for j in range(TOP_K)]
            den = sv[0]
            for j in range(1, TOP_K):
                den = den + sv[j]
            fac = SCALING / (den + 1e-20)
            obase = (g0 * UNROLL_T + u) * L * TOP_K + iota * TOP_K
            for j in range(TOP_K):
                plsc.store_scatter(w_v, [obase + j], sv[j] * fac)
                plsc.store_scatter(i_v, [obase + j], mi[j])
                plsc.addupdate_scatter(hist_v, [mi[j], iota + u * L], ones_i)
        return 0

    lax.fori_loop(0, N_STEP, step, 0)

    pltpu.sync_copy(w_v, w_hbm.at[pl.ds(base * TOP_K, TPW * TOP_K)])
    pltpu.sync_copy(i_v, i_hbm.at[pl.ds(base * TOP_K, TPW * TOP_K)])

    for f in range(N_EXP // L):
        rows = iota + L * f
        acc = plsc.load_gather(hist_v, [rows, zeros_i])
        for u in range(1, UNROLL_T * L):
            acc = acc + plsc.load_gather(hist_v, [rows, zeros_i + u])
        hred_v[pl.ds(L * f, L)] = acc
    pltpu.sync_copy(hred_v, h_hbm.at[wid])


def _hist_sum_body(h_ref, o_ref):
    o_ref[...] = jnp.sum(h_ref[...], axis=0)


_hist_sum = pl.pallas_call(
    _hist_sum_body,
    out_shape=jax.ShapeDtypeStruct((N_EXP,), jnp.int32),
)


def kernel(logits, e_score_correction_bias):
    w_flat, i_flat, h_part = _router(logits.reshape(-1), e_score_correction_bias)
    topk_weight = w_flat.reshape(N_TOK, TOP_K)
    topk_idx = i_flat.reshape(N_TOK, TOP_K)
    tokens_per_expert = _hist_sum(h_part)
    return (logits, topk_weight, topk_idx, tokens_per_expert)


# speedup vs baseline: 1.0912x; 1.0016x over previous
import functools

import jax
import jax.numpy as jnp
from jax import lax
from jax.experimental import pallas as pl
from jax.experimental.pallas import tpu as pltpu
from jax.experimental.pallas import tpu_sc as plsc

TOP_K = 8
N_EXP = 64
SCALING = 2.5
N_TOK = 32768
NC, NS, L = 2, 16, 16
NW = NC * NS
TPW = N_TOK // NW
UNROLL_T = 2
EXPERT_UNROLL = 1
N_STEP = TPW // (L * UNROLL_T)

_mesh = plsc.VectorSubcoreMesh(core_axis_name="c", subcore_axis_name="s")


@functools.partial(
    pl.kernel,
    out_type=(
        jax.ShapeDtypeStruct((N_TOK, TOP_K), jnp.float32),
        jax.ShapeDtypeStruct((N_TOK, TOP_K), jnp.int32),
        jax.ShapeDtypeStruct((NW, N_EXP), jnp.int32),
    ),
    mesh=_mesh,
    compiler_params=pltpu.CompilerParams(
        needs_layout_passes=False, use_tc_tiling_on_sc=False),
    scratch_types=(
        pltpu.VMEM((TPW, N_EXP), jnp.float32),
        pltpu.VMEM((TPW, TOP_K), jnp.float32),
        pltpu.VMEM((TPW, TOP_K), jnp.int32),
        pltpu.VMEM((N_EXP,), jnp.float32),
        pltpu.VMEM((N_EXP, UNROLL_T * L), jnp.int32),
        pltpu.VMEM((N_EXP,), jnp.int32),
    ),
)
def _router(logits_hbm, bias_hbm, w_hbm, i_hbm, h_hbm,
            x_v, w_v, i_v, b_v, hist_v, hred_v):
    c = lax.axis_index("c")
    s = lax.axis_index("s")
    wid = s * NC + c
    base = wid * TPW

    iota = jnp.arange(L, dtype=jnp.int32)
    zeros_i = jnp.zeros((L,), jnp.int32)
    ones_i = jnp.ones((L,), jnp.int32)

    pltpu.sync_copy(bias_hbm, b_v)
    pltpu.sync_copy(logits_hbm.at[pl.ds(base, TPW)], x_v)

    def _zero_body(e, _):
        for u in range(UNROLL_T):
            hist_v[e, pl.ds(u * L, L)] = zeros_i
        return 0
    lax.fori_loop(0, N_EXP, _zero_body, 0)

    neg_inf = jnp.full((L,), -jnp.inf, jnp.float32)

    def step(g0, _):
        toks = [(g0 * UNROLL_T + u) * L + iota for u in range(UNROLL_T)]

        def expert_body(eb, carry):
            ms, mis = carry
            new_ms = [list(ms[u]) for u in range(UNROLL_T)]
            new_mis = [list(mis[u]) for u in range(UNROLL_T)]
            for eu in range(EXPERT_UNROLL):
                e = eb * EXPERT_UNROLL + eu
                e_splat = zeros_i + e
                bias_e = plsc.load_gather(b_v, [e_splat])
                for u in range(UNROLL_T):
                    x = plsc.load_gather(x_v, [toks[u], e_splat])
                    v = 1.0 / (1.0 + jnp.exp(-x)) + bias_e
                    vi = e_splat
                    m = new_ms[u]
                    mi = new_mis[u]
                    for j in range(TOP_K):
                        b = v > m[j]
                        m[j], v = jnp.where(b, v, m[j]), jnp.where(b, m[j], v)
                        mi[j], vi = jnp.where(b, vi, mi[j]), jnp.where(b, mi[j], vi)
            return (tuple(tuple(r) for r in new_ms),
                    tuple(tuple(r) for r in new_mis))

        init = (
            tuple(tuple(neg_inf for _ in range(TOP_K)) for _ in range(UNROLL_T)),
            tuple(tuple(zeros_i for _ in range(TOP_K)) for _ in range(UNROLL_T)),
        )
        ms, mis = lax.fori_loop(0, N_EXP // EXPERT_UNROLL, expert_body, init)

        for u in range(UNROLL_T):
            m, mi = ms[u], mis[u]
            sv = [m[j] - plsc.load_gather(b_v, [mi[j]]) for j in range(TOP_K)]
            den = sv[0]
            for j in range(1, TOP_K):
                den = den + sv[j]
            fac = SCALING / (den + 1e-20)
            for j in range(TOP_K):
                jcol = zeros_i + j
                plsc.store_scatter(w_v, [toks[u], jcol], sv[j] * fac)
                plsc.store_scatter(i_v, [toks[u], jcol], mi[j])
                plsc.addupdate_scatter(hist_v, [mi[j], iota + u * L], ones_i)
        return 0

    lax.fori_loop(0, N_STEP, step, 0)

    pltpu.sync_copy(w_v, w_hbm.at[pl.ds(base, TPW)])
    pltpu.sync_copy(i_v, i_hbm.at[pl.ds(base, TPW)])

    for f in range(N_EXP // L):
        rows = iota + L * f
        acc = plsc.load_gather(hist_v, [rows, zeros_i])
        for u in range(1, UNROLL_T * L):
            acc = acc + plsc.load_gather(hist_v, [rows, zeros_i + u])
        hred_v[pl.ds(L * f, L)] = acc
    pltpu.sync_copy(hred_v, h_hbm.at[wid])


def _hist_sum_body(h_ref, o_ref):
    o_ref[...] = jnp.sum(h_ref[...], axis=0)


_hist_sum = pl.pallas_call(
    _hist_sum_body,
    out_shape=jax.ShapeDtypeStruct((N_EXP,), jnp.int32),
)


def kernel(logits, e_score_correction_bias):
    topk_weight, topk_idx, h_part = _router(logits, e_score_correction_bias)
    tokens_per_expert = _hist_sum(h_part)
    return (logits, topk_weight, topk_idx, tokens_per_expert)
